# baseline (speedup 1.0000x reference)
import functools

import jax
import jax.numpy as jnp
from jax import lax
from jax.experimental import pallas as pl
from jax.experimental.pallas import tpu as pltpu
from jax.experimental.pallas import tpu_sc as plsc

N = 10000
NFEAT = 256
NHID = 512
CW = 128
NCHUNK = NHID // CW
NC = 2
NS = 16
EDGE_CHUNK = 96
NJ = 112
EPT = NJ * EDGE_CHUNK
PADE = NS * EPT
ACC_ROWS = 10240
RPT = ACC_ROWS // NS
TRASH = N
RB = 400
NRB = N // RB

_mesh = plsc.VectorSubcoreMesh(core_axis_name="c", subcore_axis_name="s")


def _deg_body(dst32, out, acc, ones_v, zbuf, idx_v):
    c = lax.axis_index("c")
    s = lax.axis_index("s")
    w = c * NS + s

    @pl.loop(0, 128)
    def _fill(r):
        zbuf[r, :] = jnp.zeros((16,), jnp.float32)

    @pl.loop(0, EDGE_CHUNK)
    def _fill1(r):
        ones_v[r, :] = jnp.full((16,), 1.0, jnp.float32)

    @pl.loop(0, RPT // 128)
    def _zero(z):
        pltpu.sync_copy(zbuf, acc.at[pl.ds(s * RPT + z * 128, 128)])

    plsc.subcore_barrier()
    pltpu.sync_copy(dst32.at[w], idx_v)

    @pl.loop(0, NJ // 2)
    def _scat(j):
        pltpu.sync_copy(ones_v, acc.at[idx_v.at[j]], add=True)

    plsc.subcore_barrier()

    @pl.when(s == 0)
    def _drain():
        pltpu.sync_copy(acc, out.at[c])


_deg = pl.kernel(
    _deg_body,
    out_type=jax.ShapeDtypeStruct((NC, ACC_ROWS, 16), jnp.float32),
    mesh=_mesh,
    scratch_types=[
        pltpu.VMEM_SHARED((ACC_ROWS, 16), jnp.float32),
        pltpu.VMEM((EDGE_CHUNK, 16), jnp.float32),
        pltpu.VMEM((128, 16), jnp.float32),
        pltpu.VMEM((NJ // 2, EDGE_CHUNK), jnp.int32),
    ],
)


def _prop_body(gflat, src2, dst16, out, acc, idx_s, idx_d, buf0, buf1,
               sem0, sem1, sem2, sem3):
    c = lax.axis_index("c")
    s = lax.axis_index("s")

    pltpu.sync_copy(dst16.at[s], idx_d)

    @pl.loop(0, 64)
    def _fillr(r):
        @pl.loop(0, CW // 16)
        def _fillq(q):
            buf0[r, pl.ds(q * 16, 16)] = jnp.zeros((16,), jnp.float32)

    @pl.loop(0, RPT // 64)
    def _zero(z):
        pltpu.sync_copy(buf0.at[pl.ds(0, 64)],
                        acc.at[pl.ds(s * RPT + z * 64, 64)])

    plsc.subcore_barrier()

    for h in range(2):
        hb = h * (NJ // 2)
        pltpu.sync_copy(src2.at[c, 2 * s + h], idx_s)
        pltpu.async_copy(gflat.at[idx_s.at[0]], buf0, sem0)
        pltpu.async_copy(gflat.at[idx_s.at[1]], buf1, sem1)

        @pl.loop(0, NJ // 4)
        def _edge(jj):
            j0 = 2 * jj
            pltpu.make_async_copy(gflat.at[idx_s.at[j0]], buf0, sem0).wait()
            pltpu.async_copy(buf0, acc.at[idx_d.at[hb + j0]], sem2, add=True)
            pltpu.make_async_copy(
                gflat.at[idx_s.at[j0 + 1]], buf1, sem1).wait()
            pltpu.async_copy(buf1, acc.at[idx_d.at[hb + j0 + 1]], sem3,
                             add=True)

            @pl.when(jj + 1 < NJ // 4)
            def _next():
                pltpu.make_async_copy(
                    buf0, acc.at[idx_d.at[hb + j0]], sem2).wait()
                pltpu.async_copy(gflat.at[idx_s.at[j0 + 2]], buf0, sem0)
                pltpu.make_async_copy(
                    buf1, acc.at[idx_d.at[hb + j0 + 1]], sem3).wait()
                pltpu.async_copy(gflat.at[idx_s.at[j0 + 3]], buf1, sem1)

        pltpu.make_async_copy(buf0, acc.at[idx_d.at[hb]], sem2).wait()
        pltpu.make_async_copy(buf1, acc.at[idx_d.at[hb + 1]], sem3).wait()

    plsc.subcore_barrier()

    @pl.when(s == 0)
    def _drain():
        pltpu.sync_copy(acc.at[pl.ds(0, N)], out.at[c])

    plsc.subcore_barrier()


_prop = pl.kernel(
    _prop_body,
    out_type=jax.ShapeDtypeStruct((2, N, CW), jnp.float32),
    mesh=_mesh,
    scratch_types=[
        pltpu.VMEM_SHARED((ACC_ROWS, CW), jnp.float32),
        pltpu.VMEM((NJ // 2, EDGE_CHUNK), jnp.int32),
        pltpu.VMEM((NJ, EDGE_CHUNK), jnp.int32),
        pltpu.VMEM((EDGE_CHUNK, CW), jnp.float32),
        pltpu.VMEM((EDGE_CHUNK, CW), jnp.float32),
        pltpu.SemaphoreType.DMA,
        pltpu.SemaphoreType.DMA,
        pltpu.SemaphoreType.DMA,
        pltpu.SemaphoreType.DMA,
    ],
)


def _dinv_body(degp, out):
    d = degp[0, :, 0:1] + degp[1, :, 0:1]
    out[...] = lax.rsqrt(d)


def _dinv(degp):
    return pl.pallas_call(
        _dinv_body,
        out_shape=jax.ShapeDtypeStruct((ACC_ROWS, 1), jnp.float32),
    )(degp)


def _mm1_body(x, w, dinv, o):
    y = jnp.dot(x[...], w[...], preferred_element_type=jnp.float32)
    o[...] = (y * dinv[...])[None]


def _mm1(x, w1h, dinv):
    return pl.pallas_call(
        _mm1_body,
        grid=(NRB, 2),
        in_specs=[
            pl.BlockSpec((RB, NFEAT), lambda i, ko: (i, 0)),
            pl.BlockSpec((NFEAT, CW), lambda i, ko: (0, ko)),
            pl.BlockSpec((RB, 1), lambda i, ko: (i, 0)),
        ],
        out_specs=pl.BlockSpec((1, RB, CW), lambda i, ko: (ko, i, 0)),
        out_shape=jax.ShapeDtypeStruct((2, N, CW), jnp.float32),
    )(x, w1h, dinv)


def _mmp_body(pa, pb, w, b, dinv, o):
    xc = jnp.concatenate([pa[0], pa[1], pb[0], pb[1]], axis=1)
    xc = xc * dinv[...] + b[...]
    y = jnp.dot(xc, w[...], preferred_element_type=jnp.float32)
    o[...] = (y * dinv[...])[None]


def _mmp(pa, pb, wh, b_prev, dinv):
    return pl.pallas_call(
        _mmp_body,
        grid=(NRB, 2),
        in_specs=[
            pl.BlockSpec((2, RB, CW), lambda i, ko: (0, i, 0)),
            pl.BlockSpec((2, RB, CW), lambda i, ko: (0, i, 0)),
            pl.BlockSpec((NHID, CW), lambda i, ko: (0, ko)),
            pl.BlockSpec((1, NHID), lambda i, ko: (0, 0)),
            pl.BlockSpec((RB, 1), lambda i, ko: (i, 0)),
        ],
        out_specs=pl.BlockSpec((1, RB, CW), lambda i, ko: (ko, i, 0)),
        out_shape=jax.ShapeDtypeStruct((2, N, CW), jnp.float32),
    )(pa, pb, wh, b_prev, dinv)


def _fin_body(pa, pb, b, dinv, o):
    xc = jnp.concatenate([pa[0], pa[1], pb[0], pb[1]], axis=1)
    o[...] = xc * dinv[...] + b[...]


def _fin(pa, pb, b3, dinv):
    return pl.pallas_call(
        _fin_body,
        grid=(NRB,),
        in_specs=[
            pl.BlockSpec((2, RB, CW), lambda i: (0, i, 0)),
            pl.BlockSpec((2, RB, CW), lambda i: (0, i, 0)),
            pl.BlockSpec((1, NHID), lambda i: (0, 0)),
            pl.BlockSpec((RB, 1), lambda i: (i, 0)),
        ],
        out_specs=pl.BlockSpec((RB, NHID), lambda i: (i, 0)),
        out_shape=jax.ShapeDtypeStruct((N, NHID), jnp.float32),
    )(pa, pb, b3, dinv)


@jax.jit
def kernel(x, edge_index, W1, b1, W2, b2, W3, b3):
    ei = edge_index.astype(jnp.int32)
    src, dst = ei[0], ei[1]
    e = src.shape[0]
    npad = PADE - e - N
    loop = jnp.arange(N, dtype=jnp.int32)
    srcp = jnp.concatenate([src, loop, jnp.zeros((npad,), jnp.int32)])
    dstp = jnp.concatenate([dst, loop, jnp.full((npad,), TRASH, jnp.int32)])
    src16 = srcp.reshape(NS, NJ, EDGE_CHUNK)
    src2 = src16[None] + (jnp.arange(2, dtype=jnp.int32) * N)[:, None, None, None]
    src2 = src2.reshape(2, NS * 2, NJ // 2, EDGE_CHUNK)
    dst16 = dstp.reshape(NS, NJ, EDGE_CHUNK)
    dst32 = dstp.reshape(NC * NS, NJ // 2, EDGE_CHUNK)

    degp = _deg(dst32)
    dinv = _dinv(degp)[:N]

    b1r = b1.reshape(1, NHID)
    b2r = b2.reshape(1, NHID)
    b3r = b3.reshape(1, NHID)

    def layer(mm, *args):
        ga = mm(args[0][:, :2 * CW])
        gb = mm(args[0][:, 2 * CW:])
        pa = _prop(ga.reshape(2 * N, CW), src2, dst16)
        pb = _prop(gb.reshape(2 * N, CW), src2, dst16)
        return pa, pb

    pa, pb = layer(lambda wh: _mm1(x, wh, dinv), W1)
    pa, pb = layer(lambda wh: _mmp(pa, pb, wh, b1r, dinv), W2)
    pa, pb = layer(lambda wh: _mmp(pa, pb, wh, b2r, dinv), W3)
    return _fin(pa, pb, b3r, dinv)


# speedup vs baseline: 1.0303x; 1.0303x over previous
import functools

import jax
import jax.numpy as jnp
from jax import lax
from jax.experimental import pallas as pl
from jax.experimental.pallas import tpu as pltpu
from jax.experimental.pallas import tpu_sc as plsc

N = 10000
NFEAT = 256
NHID = 512
CW = 128
NCHUNK = NHID // CW
NC = 2
NS = 16
EDGE_CHUNK = 96
NJ = 112
EPT = NJ * EDGE_CHUNK
PADE = NS * EPT
ACC_ROWS = 10240
RPT = ACC_ROWS // NS
TRASH = N
RB = 400
NRB = N // RB

_mesh = plsc.VectorSubcoreMesh(core_axis_name="c", subcore_axis_name="s")


def _deg_body(dst32, out, acc, ones_v, zbuf, idx_v):
    c = lax.axis_index("c")
    s = lax.axis_index("s")
    w = c * NS + s

    @pl.loop(0, 128)
    def _fill(r):
        zbuf[r, :] = jnp.zeros((16,), jnp.float32)

    @pl.loop(0, EDGE_CHUNK)
    def _fill1(r):
        ones_v[r, :] = jnp.full((16,), 1.0, jnp.float32)

    @pl.loop(0, RPT // 128)
    def _zero(z):
        pltpu.sync_copy(zbuf, acc.at[pl.ds(s * RPT + z * 128, 128)])

    plsc.subcore_barrier()
    pltpu.sync_copy(dst32.at[w], idx_v)

    @pl.loop(0, NJ // 2)
    def _scat(j):
        pltpu.sync_copy(ones_v, acc.at[idx_v.at[j]], add=True)

    plsc.subcore_barrier()

    @pl.when(s == 0)
    def _drain():
        pltpu.sync_copy(acc, out.at[c])


_deg = pl.kernel(
    _deg_body,
    out_type=jax.ShapeDtypeStruct((NC, ACC_ROWS, 16), jnp.float32),
    mesh=_mesh,
    scratch_types=[
        pltpu.VMEM_SHARED((ACC_ROWS, 16), jnp.float32),
        pltpu.VMEM((EDGE_CHUNK, 16), jnp.float32),
        pltpu.VMEM((128, 16), jnp.float32),
        pltpu.VMEM((NJ // 2, EDGE_CHUNK), jnp.int32),
    ],
)


def _prop_body(gflat, src2, dst16, out, acc, idx_s, idx_d, buf0, buf1,
               sem0, sem1):
    c = lax.axis_index("c")
    s = lax.axis_index("s")

    pltpu.sync_copy(dst16.at[s], idx_d)

    @pl.loop(0, 64)
    def _fillr(r):
        @pl.loop(0, CW // 16)
        def _fillq(q):
            buf0[r, pl.ds(q * 16, 16)] = jnp.zeros((16,), jnp.float32)

    @pl.loop(0, RPT // 64)
    def _zero(z):
        pltpu.sync_copy(buf0.at[pl.ds(0, 64)],
                        acc.at[pl.ds(s * RPT + z * 64, 64)])

    plsc.subcore_barrier()

    for h in range(2):
        hb = h * (NJ // 2)
        pltpu.sync_copy(src2.at[c, 2 * s + h], idx_s)
        pltpu.async_copy(gflat.at[idx_s.at[0]], buf0, sem0)

        @pl.loop(0, NJ // 4)
        def _edge(jj):
            j0 = 2 * jj
            pltpu.make_async_copy(gflat.at[idx_s.at[j0]], buf0, sem0).wait()
            pltpu.async_copy(gflat.at[idx_s.at[j0 + 1]], buf1, sem1)
            pltpu.sync_copy(buf0, acc.at[idx_d.at[hb + j0]], add=True)
            pltpu.make_async_copy(
                gflat.at[idx_s.at[j0 + 1]], buf1, sem1).wait()

            @pl.when(jj + 1 < NJ // 4)
            def _next():
                pltpu.async_copy(gflat.at[idx_s.at[j0 + 2]], buf0, sem0)

            pltpu.sync_copy(buf1, acc.at[idx_d.at[hb + j0 + 1]], add=True)

    plsc.subcore_barrier()

    @pl.when(s < NS - 1)
    def _drain():
        pltpu.sync_copy(acc.at[pl.ds(s * RPT, RPT)],
                        out.at[pl.ds(c * N + s * RPT, RPT)])

    @pl.when(s == NS - 1)
    def _drain_last():
        pltpu.sync_copy(acc.at[pl.ds((NS - 1) * RPT, N - (NS - 1) * RPT)],
                        out.at[pl.ds(c * N + (NS - 1) * RPT,
                                     N - (NS - 1) * RPT)])

    plsc.subcore_barrier()


_prop = pl.kernel(
    _prop_body,
    out_type=jax.ShapeDtypeStruct((2 * N, CW), jnp.float32),
    mesh=_mesh,
    scratch_types=[
        pltpu.VMEM_SHARED((ACC_ROWS, CW), jnp.float32),
        pltpu.VMEM((NJ // 2, EDGE_CHUNK), jnp.int32),
        pltpu.VMEM((NJ, EDGE_CHUNK), jnp.int32),
        pltpu.VMEM((EDGE_CHUNK, CW), jnp.float32),
        pltpu.VMEM((EDGE_CHUNK, CW), jnp.float32),
        pltpu.SemaphoreType.DMA,
        pltpu.SemaphoreType.DMA,
    ],
)


def _dinv_body(degp, out):
    d = degp[0, :, 0:1] + degp[1, :, 0:1]
    out[...] = lax.rsqrt(d)


def _dinv(degp):
    return pl.pallas_call(
        _dinv_body,
        out_shape=jax.ShapeDtypeStruct((ACC_ROWS, 1), jnp.float32),
    )(degp)


def _mm1_body(x, w, dinv, o):
    y = jnp.dot(x[...], w[...], preferred_element_type=jnp.float32)
    o[...] = (y * dinv[...])[None]


def _mm1(x, w1h, dinv):
    return pl.pallas_call(
        _mm1_body,
        grid=(NRB, 2),
        in_specs=[
            pl.BlockSpec((RB, NFEAT), lambda i, ko: (i, 0)),
            pl.BlockSpec((NFEAT, CW), lambda i, ko: (0, ko)),
            pl.BlockSpec((RB, 1), lambda i, ko: (i, 0)),
        ],
        out_specs=pl.BlockSpec((1, RB, CW), lambda i, ko: (ko, i, 0)),
        out_shape=jax.ShapeDtypeStruct((2, N, CW), jnp.float32),
    )(x, w1h, dinv)


def _mmp_body(pa, pb, w, b, dinv, o):
    xc = jnp.concatenate([pa[0], pa[1], pb[0], pb[1]], axis=1)
    xc = xc * dinv[...] + b[...]
    y = jnp.dot(xc, w[...], preferred_element_type=jnp.float32)
    o[...] = (y * dinv[...])[None]


def _mmp(pa, pb, wh, b_prev, dinv):
    return pl.pallas_call(
        _mmp_body,
        grid=(NRB, 2),
        in_specs=[
            pl.BlockSpec((2, RB, CW), lambda i, ko: (0, i, 0)),
            pl.BlockSpec((2, RB, CW), lambda i, ko: (0, i, 0)),
            pl.BlockSpec((NHID, CW), lambda i, ko: (0, ko)),
            pl.BlockSpec((1, NHID), lambda i, ko: (0, 0)),
            pl.BlockSpec((RB, 1), lambda i, ko: (i, 0)),
        ],
        out_specs=pl.BlockSpec((1, RB, CW), lambda i, ko: (ko, i, 0)),
        out_shape=jax.ShapeDtypeStruct((2, N, CW), jnp.float32),
    )(pa, pb, wh, b_prev, dinv)


def _fin_body(pa, pb, b, dinv, o):
    xc = jnp.concatenate([pa[0], pa[1], pb[0], pb[1]], axis=1)
    o[...] = xc * dinv[...] + b[...]


def _fin(pa, pb, b3, dinv):
    return pl.pallas_call(
        _fin_body,
        grid=(NRB,),
        in_specs=[
            pl.BlockSpec((2, RB, CW), lambda i: (0, i, 0)),
            pl.BlockSpec((2, RB, CW), lambda i: (0, i, 0)),
            pl.BlockSpec((1, NHID), lambda i: (0, 0)),
            pl.BlockSpec((RB, 1), lambda i: (i, 0)),
        ],
        out_specs=pl.BlockSpec((RB, NHID), lambda i: (i, 0)),
        out_shape=jax.ShapeDtypeStruct((N, NHID), jnp.float32),
    )(pa, pb, b3, dinv)


@jax.jit
def kernel(x, edge_index, W1, b1, W2, b2, W3, b3):
    ei = edge_index.astype(jnp.int32)
    src, dst = ei[0], ei[1]
    e = src.shape[0]
    npad = PADE - e - N
    loop = jnp.arange(N, dtype=jnp.int32)
    srcp = jnp.concatenate([src, loop, jnp.zeros((npad,), jnp.int32)])
    dstp = jnp.concatenate([dst, loop, jnp.full((npad,), TRASH, jnp.int32)])
    src16 = srcp.reshape(NS, NJ, EDGE_CHUNK)
    src2 = src16[None] + (jnp.arange(2, dtype=jnp.int32) * N)[:, None, None, None]
    src2 = src2.reshape(2, NS * 2, NJ // 2, EDGE_CHUNK)
    dst16 = dstp.reshape(NS, NJ, EDGE_CHUNK)
    dst32 = dstp.reshape(NC * NS, NJ // 2, EDGE_CHUNK)

    degp = _deg(dst32)
    dinv = _dinv(degp)[:N]

    b1r = b1.reshape(1, NHID)
    b2r = b2.reshape(1, NHID)
    b3r = b3.reshape(1, NHID)

    def layer(mm, *args):
        ga = mm(args[0][:, :2 * CW])
        gb = mm(args[0][:, 2 * CW:])
        pa = _prop(ga.reshape(2 * N, CW), src2, dst16).reshape(2, N, CW)
        pb = _prop(gb.reshape(2 * N, CW), src2, dst16).reshape(2, N, CW)
        return pa, pb

    pa, pb = layer(lambda wh: _mm1(x, wh, dinv), W1)
    pa, pb = layer(lambda wh: _mmp(pa, pb, wh, b1r, dinv), W2)
    pa, pb = layer(lambda wh: _mmp(pa, pb, wh, b2r, dinv), W3)
    return _fin(pa, pb, b3r, dinv)
